# baseline (speedup 1.0000x reference)
import functools

import jax
import jax.numpy as jnp
from jax import lax
from jax.experimental import pallas as pl
from jax.experimental.pallas import tpu as pltpu
from jax.experimental.pallas import tpu_sc as plsc

N_NODES = 10000
N_EDGES = 320000
D = 128

NC = 2
NS = 16
NW = NC * NS
C = 128
G0 = 56
G1 = 104
TOT_CHUNKS = NS * (G0 + G1)
E_PAD = TOT_CHUNKS * C
ROWS_PER_TILE = 640
N_PAD = NS * ROWS_PER_TILE


def _agg_body(x_hbm, srcs_hbm, dsts_hbm, out_hbm,
              src_v, dst_v, rows_v, accum, gsem, ssem):
  cid = lax.axis_index("c")
  sid = lax.axis_index("s")

  zeros16 = jnp.zeros((16,), jnp.float32)

  def _zrow(i, _):
    for k in range(8):
      rows_v[i, pl.ds(k * 16, 16)] = zeros16
    return 0

  lax.fori_loop(0, 128, _zrow, 0)
  for t in range(ROWS_PER_TILE // 128):
    pltpu.sync_copy(rows_v, accum.at[pl.ds(sid * ROWS_PER_TILE + t * 128, 128)])
  plsc.subcore_barrier()

  def _chunk(j, _):
    pltpu.async_copy(x_hbm.at[src_v.at[j]], rows_v, gsem).wait()
    pltpu.async_copy(rows_v, accum.at[dst_v.at[j]], ssem, add=True).wait()
    return 0

  @pl.when(cid == 0)
  def _():
    pltpu.sync_copy(srcs_hbm.at[pl.ds(sid * G0, G0)], src_v.at[pl.ds(0, G0)])
    pltpu.sync_copy(dsts_hbm.at[pl.ds(sid * G0, G0)], dst_v.at[pl.ds(0, G0)])
    lax.fori_loop(0, G0, _chunk, 0)

  @pl.when(cid == 1)
  def _():
    base = NS * G0 + sid * G1
    pltpu.sync_copy(srcs_hbm.at[pl.ds(base, G1)], src_v.at[pl.ds(0, G1)])
    pltpu.sync_copy(dsts_hbm.at[pl.ds(base, G1)], dst_v.at[pl.ds(0, G1)])
    lax.fori_loop(0, G1, _chunk, 0)

  plsc.subcore_barrier()

  for t in range(ROWS_PER_TILE // 128):
    base = sid * ROWS_PER_TILE + t * 128
    pltpu.sync_copy(accum.at[pl.ds(base, 128)], out_hbm.at[cid, pl.ds(base, 128)])


_agg = pl.kernel(
    _agg_body,
    out_type=jax.ShapeDtypeStruct((NC, N_PAD, D), jnp.float32),
    mesh=plsc.VectorSubcoreMesh(core_axis_name="c", subcore_axis_name="s"),
    scratch_types=[
        pltpu.VMEM((max(G0, G1), C), jnp.int32),
        pltpu.VMEM((max(G0, G1), C), jnp.int32),
        pltpu.VMEM((C, D), jnp.float32),
        pltpu.VMEM_SHARED((N_PAD, D), jnp.float32),
        pltpu.SemaphoreType.DMA,
        pltpu.SemaphoreType.DMA,
    ],
)


def _mlp_body(x_ref, p0_ref, p1_ref, w1_ref, b1_ref, w2_ref, b2_ref, o_ref):
  h = x_ref[...] + p0_ref[0] + p1_ref[0]
  h = jnp.dot(h, w1_ref[...], preferred_element_type=jnp.float32) + b1_ref[...]
  h = jnp.maximum(h, 0.0)
  o_ref[...] = (
      jnp.dot(h, w2_ref[...], preferred_element_type=jnp.float32) + b2_ref[...]
  )


def _mlp(x, partials, w1t, b1, w2t, b2):
  R = 2000
  grid = (N_NODES // R,)
  return pl.pallas_call(
      _mlp_body,
      grid=grid,
      in_specs=[
          pl.BlockSpec((R, D), lambda i: (i, 0)),
          pl.BlockSpec((1, R, D), lambda i: (0, i, 0)),
          pl.BlockSpec((1, R, D), lambda i: (1, i, 0)),
          pl.BlockSpec((D, D), lambda i: (0, 0)),
          pl.BlockSpec((1, D), lambda i: (0, 0)),
          pl.BlockSpec((D, D), lambda i: (0, 0)),
          pl.BlockSpec((1, D), lambda i: (0, 0)),
      ],
      out_specs=pl.BlockSpec((R, D), lambda i: (i, 0)),
      out_shape=jax.ShapeDtypeStruct((N_NODES, D), jnp.float32),
  )(x, partials, partials, w1t, b1, w2t, b2)


@jax.jit
def kernel(x, edge_index, W1, b1, W2, b2):
  src = edge_index[0].astype(jnp.int32)
  dst = edge_index[1].astype(jnp.int32)
  pad = E_PAD - N_EDGES
  src = jnp.concatenate([src, jnp.zeros((pad,), jnp.int32)])
  dst = jnp.concatenate([dst, jnp.full((pad,), N_NODES, jnp.int32)])
  srcs = src.reshape(TOT_CHUNKS, C)
  dsts = dst.reshape(TOT_CHUNKS, C)

  partials = _agg(x, srcs, dsts)

  return _mlp(x, partials, W1.T, b1.reshape(1, D), W2.T, b2.reshape(1, D))


# speedup vs baseline: 1.6238x; 1.6238x over previous
import functools

import jax
import jax.numpy as jnp
from jax import lax
from jax.experimental import pallas as pl
from jax.experimental.pallas import tpu as pltpu
from jax.experimental.pallas import tpu_sc as plsc

N_NODES = 10000
N_EDGES = 320000
D = 128

NC = 2
NS = 16
NW = NC * NS
C = 128
G = 79
E_PAD = NW * G * C
ROWS_PER_TILE = 640
N_PAD = NS * ROWS_PER_TILE


def _agg_body(x_hbm, srcs_hbm, dsts_hbm, out_hbm,
              src_v, dst_v, rows_v, accum, gsem, ssem):
  cid = lax.axis_index("c")
  sid = lax.axis_index("s")
  wid = sid * NC + cid

  zeros16 = jnp.zeros((16,), jnp.float32)

  def _zrow(i, _):
    for k in range(8):
      rows_v[i, pl.ds(k * 16, 16)] = zeros16
    return 0

  lax.fori_loop(0, 128, _zrow, 0)
  for t in range(ROWS_PER_TILE // 128):
    pltpu.sync_copy(rows_v, accum.at[pl.ds(sid * ROWS_PER_TILE + t * 128, 128)])
  plsc.subcore_barrier()

  pltpu.sync_copy(srcs_hbm.at[wid], src_v)
  pltpu.sync_copy(dsts_hbm.at[wid], dst_v)

  def _chunk(j, _):
    pltpu.async_copy(x_hbm.at[src_v.at[j]], rows_v, gsem).wait()
    pltpu.async_copy(rows_v, accum.at[dst_v.at[j]], ssem, add=True).wait()
    return 0

  lax.fori_loop(0, G, _chunk, 0)
  plsc.subcore_barrier()

  for t in range(ROWS_PER_TILE // 128):
    base = sid * ROWS_PER_TILE + t * 128
    pltpu.sync_copy(accum.at[pl.ds(base, 128)], out_hbm.at[cid, pl.ds(base, 128)])


_agg = pl.kernel(
    _agg_body,
    out_type=jax.ShapeDtypeStruct((NC, N_PAD, D), jnp.float32),
    mesh=plsc.VectorSubcoreMesh(core_axis_name="c", subcore_axis_name="s"),
    scratch_types=[
        pltpu.VMEM((G, C), jnp.int32),
        pltpu.VMEM((G, C), jnp.int32),
        pltpu.VMEM((C, D), jnp.float32),
        pltpu.VMEM_SHARED((N_PAD, D), jnp.float32),
        pltpu.SemaphoreType.DMA,
        pltpu.SemaphoreType.DMA,
    ],
)


def _mlp_body(x_ref, p0_ref, p1_ref, w1_ref, b1_ref, w2_ref, b2_ref, o_ref):
  h = x_ref[...] + p0_ref[0] + p1_ref[0]
  h = jnp.dot(h, w1_ref[...], preferred_element_type=jnp.float32) + b1_ref[...]
  h = jnp.maximum(h, 0.0)
  o_ref[...] = (
      jnp.dot(h, w2_ref[...], preferred_element_type=jnp.float32) + b2_ref[...]
  )


def _mlp(x, partials, w1t, b1, w2t, b2):
  R = 2000
  grid = (N_NODES // R,)
  return pl.pallas_call(
      _mlp_body,
      grid=grid,
      in_specs=[
          pl.BlockSpec((R, D), lambda i: (i, 0)),
          pl.BlockSpec((1, R, D), lambda i: (0, i, 0)),
          pl.BlockSpec((1, R, D), lambda i: (1, i, 0)),
          pl.BlockSpec((D, D), lambda i: (0, 0)),
          pl.BlockSpec((1, D), lambda i: (0, 0)),
          pl.BlockSpec((D, D), lambda i: (0, 0)),
          pl.BlockSpec((1, D), lambda i: (0, 0)),
      ],
      out_specs=pl.BlockSpec((R, D), lambda i: (i, 0)),
      out_shape=jax.ShapeDtypeStruct((N_NODES, D), jnp.float32),
  )(x, partials, partials, w1t, b1, w2t, b2)


@jax.jit
def kernel(x, edge_index, W1, b1, W2, b2):
  src = edge_index[0].astype(jnp.int32)
  dst = edge_index[1].astype(jnp.int32)
  pad = E_PAD - N_EDGES
  src = jnp.concatenate([src, jnp.zeros((pad,), jnp.int32)])
  dst = jnp.concatenate([dst, jnp.full((pad,), N_NODES, jnp.int32)])
  srcs = src.reshape(NW, G, C)
  dsts = dst.reshape(NW, G, C)

  partials = _agg(x, srcs, dsts)

  return _mlp(x, partials, W1.T, b1.reshape(1, D), W2.T, b2.reshape(1, D))
